# baseline (speedup 1.0000x reference)
import jax
import jax.numpy as jnp
from jax.experimental import pallas as pl

BATCH = 32
SEQ_LEN = 8192
N_ALLELES = 4
D_MODEL = 64


def _body(x_ref, a_ref, p_ref, o_ref):
    for i in range(2):
        emb = jax.lax.dot_general(
            x_ref[i], a_ref[...],
            dimension_numbers=(((1,), (0,)), ((), ())),
            preferred_element_type=jnp.float32,
        )
        o_ref[i] = emb + p_ref[...]


def kernel(x, allele_embedding, position_table):
    return pl.pallas_call(
        _body,
        grid=(BATCH // 2,),
        in_specs=[
            pl.BlockSpec((2, SEQ_LEN, N_ALLELES), lambda b: (b, 0, 0)),
            pl.BlockSpec((N_ALLELES, D_MODEL), lambda b: (0, 0)),
            pl.BlockSpec((SEQ_LEN, D_MODEL), lambda b: (0, 0)),
        ],
        out_specs=pl.BlockSpec((2, SEQ_LEN, D_MODEL), lambda b: (b, 0, 0)),
        out_shape=jax.ShapeDtypeStruct((BATCH, SEQ_LEN, D_MODEL), jnp.float32),
    )(x, allele_embedding, position_table)
